# baseline (speedup 1.0000x reference)
import functools

import jax
import jax.numpy as jnp
from jax import lax
from jax.experimental import pallas as pl
from jax.experimental.pallas import tpu as pltpu
from jax.experimental.pallas import tpu_sc as plsc

MIN_NORM = 1e-15
BALL_EPS = 4e-3

N = 10000
D = 128
E = 320000

NC = 2
NS = 16
NW = NC * NS
EPW = E // NW
CH = 128
KF = EPW // CH
TAIL = EPW - KF * CH
G = 6
NG = KF // G
ACC_ROWS = 10112
ZPT = ACC_ROWS // NS


def _logmap_body(x_ref, o_ref):
    x = x_ref[...]
    n = jnp.sqrt(jnp.sum(x * x, axis=-1, keepdims=True))
    n = jnp.maximum(n, MIN_NORM)
    z = jnp.clip(n, -1.0 + 1e-7, 1.0 - 1e-7)
    at = 0.5 * jnp.log((1.0 + z) / (1.0 - z))
    o_ref[...] = x * (at / n)


def _logmap(x):
    br = 1000
    return pl.pallas_call(
        _logmap_body,
        grid=(N // br,),
        in_specs=[pl.BlockSpec((br, D), lambda i: (i, 0))],
        out_specs=pl.BlockSpec((br, D), lambda i: (i, 0)),
        out_shape=jax.ShapeDtypeStruct((N, D), jnp.float32),
    )(x)


def _expproj_body(p_ref, o_ref):
    u = p_ref[0] + p_ref[1]
    n = jnp.sqrt(jnp.sum(u * u, axis=-1, keepdims=True))
    n = jnp.maximum(n, MIN_NORM)
    y = jnp.tanh(n) * u / n
    yn = jnp.sqrt(jnp.sum(y * y, axis=-1, keepdims=True))
    yn = jnp.maximum(yn, MIN_NORM)
    maxnorm = 1.0 - BALL_EPS
    o_ref[...] = jnp.where(yn > maxnorm, y / yn * maxnorm, y)


def _expproj(parts):
    br = 1000
    return pl.pallas_call(
        _expproj_body,
        grid=(N // br,),
        in_specs=[pl.BlockSpec((NC, br, D), lambda i: (0, i, 0))],
        out_specs=pl.BlockSpec((br, D), lambda i: (i, 0)),
        out_shape=jax.ShapeDtypeStruct((N, D), jnp.float32),
    )(parts)


_MESH = plsc.VectorSubcoreMesh(core_axis_name="c", subcore_axis_name="s")


@functools.partial(
    pl.kernel,
    mesh=_MESH,
    out_type=jax.ShapeDtypeStruct((NC, ACC_ROWS, D), jnp.float32),
    scratch_types=[
        pltpu.VMEM((2, G, CH), jnp.int32),
        pltpu.VMEM((2, G, CH), jnp.int32),
        pltpu.VMEM((1, TAIL), jnp.int32),
        pltpu.VMEM((1, TAIL), jnp.int32),
        pltpu.VMEM((2, CH, D), jnp.float32),
        pltpu.VMEM_SHARED((ACC_ROWS, D), jnp.float32),
        pltpu.SemaphoreType.DMA,
        pltpu.SemaphoreType.DMA,
        pltpu.SemaphoreType.DMA,
    ],
)
def _agg(xt_hbm, s_hbm, r_hbm, zeros_hbm, out_hbm, s_v, r_v, st_v, rt_v, buf,
         acc, sem, sem_i, sem_s):
    cid = lax.axis_index("c")
    sid = lax.axis_index("s")
    wid = sid * NC + cid
    base = wid * EPW

    pltpu.sync_copy(zeros_hbm.at[pl.ds(sid * ZPT, ZPT)],
                    acc.at[pl.ds(sid * ZPT, ZPT)])

    def idx_rows(g, bank, wait):
        for i in range(G):
            off = base + (g * G + i) * CH
            for hbm, vmem in ((s_hbm, s_v), (r_hbm, r_v)):
                d = pltpu.make_async_copy(hbm.at[pl.ds(off, CH)],
                                          vmem.at[bank, i], sem_i)
                if wait:
                    d.wait()
                else:
                    d.start()

    idx_rows(0, 0, wait=False)
    pltpu.async_copy(s_hbm.at[pl.ds(base + KF * CH, TAIL)], st_v.at[0], sem_i)
    pltpu.async_copy(r_hbm.at[pl.ds(base + KF * CH, TAIL)], rt_v.at[0], sem_i)
    idx_rows(0, 0, wait=True)
    pltpu.make_async_copy(s_hbm.at[pl.ds(base + KF * CH, TAIL)], st_v.at[0],
                          sem_i).wait()
    pltpu.make_async_copy(r_hbm.at[pl.ds(base + KF * CH, TAIL)], rt_v.at[0],
                          sem_i).wait()
    plsc.subcore_barrier()
    pltpu.async_copy(xt_hbm.at[s_v.at[0, 0]], buf.at[0], sem)

    def emit_group(gb, g, is_last):
        if not is_last:
            idx_rows(g + 1, 1 - gb, wait=False)
        for u in range(G):
            cb = u % 2
            pltpu.make_async_copy(xt_hbm.at[s_v.at[gb, u]], buf.at[cb],
                                  sem).wait()

            def _wait_prev():
                pltpu.make_async_copy(buf.at[1 - cb], acc.at[r_v.at[gb, u]],
                                      sem_s).wait()
            if u == 0 and gb == 0 and not is_last:
                @pl.when(g > 0)
                def _():
                    _wait_prev()
            else:
                _wait_prev()

            if u < G - 1:
                pltpu.async_copy(xt_hbm.at[s_v.at[gb, u + 1]],
                                 buf.at[1 - cb], sem)
            elif not is_last:
                idx_rows(g + 1, 1 - gb, wait=True)
                pltpu.async_copy(xt_hbm.at[s_v.at[1 - gb, 0]],
                                 buf.at[1 - cb], sem)

            pltpu.async_copy(buf.at[cb], acc.at[r_v.at[gb, u]], sem_s,
                             add=True)

    def body(g2, carry):
        emit_group(0, g2 * 2, False)
        emit_group(1, g2 * 2 + 1, False)
        return carry

    lax.fori_loop(0, NG // 2, body, jnp.int32(0))
    emit_group(0, NG - 1, True)

    pltpu.async_copy(xt_hbm.at[st_v.at[0]], buf.at[0, pl.ds(0, TAIL)],
                     sem).wait()
    pltpu.make_async_copy(buf.at[1], acc.at[r_v.at[0, G - 1]], sem_s).wait()
    pltpu.sync_copy(buf.at[0, pl.ds(0, TAIL)], acc.at[rt_v.at[0]], add=True)
    plsc.subcore_barrier()

    rbase = sid * ZPT
    pltpu.sync_copy(acc.at[pl.ds(rbase, ZPT)],
                    out_hbm.at[cid, pl.ds(rbase, ZPT)])


def kernel(x, adj):
    s = adj[0].astype(jnp.int32)
    r = adj[1].astype(jnp.int32)
    zeros = jnp.zeros((ACC_ROWS, D), jnp.float32)
    xt = _logmap(x)
    parts = _agg(xt, s, r, zeros)
    return _expproj(parts)


# speedup vs baseline: 1.0026x; 1.0026x over previous
import functools

import jax
import jax.numpy as jnp
from jax import lax
from jax.experimental import pallas as pl
from jax.experimental.pallas import tpu as pltpu
from jax.experimental.pallas import tpu_sc as plsc

MIN_NORM = 1e-15
BALL_EPS = 4e-3

N = 10000
D = 128
E = 320000

NC = 2
NS = 16
NW = NC * NS
EPW = E // NW
CH = 128
KF = EPW // CH
TAIL = EPW - KF * CH
G = 6
NG = KF // G
ACC_ROWS = 10112
ZPT = ACC_ROWS // NS


def _logmap_body(x_ref, o_ref):
    x = x_ref[...]
    n = jnp.sqrt(jnp.sum(x * x, axis=-1, keepdims=True))
    n = jnp.maximum(n, MIN_NORM)
    z = jnp.clip(n, -1.0 + 1e-7, 1.0 - 1e-7)
    at = 0.5 * jnp.log((1.0 + z) / (1.0 - z))
    o_ref[...] = x * (at / n)


def _logmap(x):
    br = 1000
    return pl.pallas_call(
        _logmap_body,
        grid=(N // br,),
        in_specs=[pl.BlockSpec((br, D), lambda i: (i, 0))],
        out_specs=pl.BlockSpec((br, D), lambda i: (i, 0)),
        out_shape=jax.ShapeDtypeStruct((N, D), jnp.float32),
    )(x)


def _expproj_body(p_ref, o_ref):
    u = p_ref[0] + p_ref[1]
    n = jnp.sqrt(jnp.sum(u * u, axis=-1, keepdims=True))
    n = jnp.maximum(n, MIN_NORM)
    y = jnp.tanh(n) * u / n
    yn = jnp.sqrt(jnp.sum(y * y, axis=-1, keepdims=True))
    yn = jnp.maximum(yn, MIN_NORM)
    maxnorm = 1.0 - BALL_EPS
    o_ref[...] = jnp.where(yn > maxnorm, y / yn * maxnorm, y)


def _expproj(parts):
    br = 1000
    return pl.pallas_call(
        _expproj_body,
        grid=(N // br,),
        in_specs=[pl.BlockSpec((NC, br, D), lambda i: (0, i, 0))],
        out_specs=pl.BlockSpec((br, D), lambda i: (i, 0)),
        out_shape=jax.ShapeDtypeStruct((N, D), jnp.float32),
    )(parts)


_MESH = plsc.VectorSubcoreMesh(core_axis_name="c", subcore_axis_name="s")


@functools.partial(
    pl.kernel,
    mesh=_MESH,
    out_type=jax.ShapeDtypeStruct((NC, ACC_ROWS, D), jnp.float32),
    scratch_types=[
        pltpu.VMEM((2, G, CH), jnp.int32),
        pltpu.VMEM((2, G, CH), jnp.int32),
        pltpu.VMEM((1, TAIL), jnp.int32),
        pltpu.VMEM((1, TAIL), jnp.int32),
        pltpu.VMEM((2, CH, D), jnp.float32),
        pltpu.VMEM_SHARED((ACC_ROWS, D), jnp.float32),
        pltpu.SemaphoreType.DMA,
        pltpu.SemaphoreType.DMA,
        pltpu.SemaphoreType.DMA,
        pltpu.SemaphoreType.DMA,
    ],
)
def _agg(xt_hbm, s_hbm, r_hbm, zeros_hbm, out_hbm, s_v, r_v, st_v, rt_v, buf,
         acc, sem, sem_i, sem_s, sem_z):
    cid = lax.axis_index("c")
    sid = lax.axis_index("s")
    wid = sid * NC + cid
    base = wid * EPW

    zero_cp = pltpu.make_async_copy(zeros_hbm.at[pl.ds(sid * ZPT, ZPT)],
                                    acc.at[pl.ds(sid * ZPT, ZPT)], sem_z)
    zero_cp.start()

    def idx_rows(g, bank, wait):
        for i in range(G):
            off = base + (g * G + i) * CH
            for hbm, vmem in ((s_hbm, s_v), (r_hbm, r_v)):
                d = pltpu.make_async_copy(hbm.at[pl.ds(off, CH)],
                                          vmem.at[bank, i], sem_i)
                if wait:
                    d.wait()
                else:
                    d.start()

    idx_rows(0, 0, wait=False)
    pltpu.async_copy(s_hbm.at[pl.ds(base + KF * CH, TAIL)], st_v.at[0], sem_i)
    pltpu.async_copy(r_hbm.at[pl.ds(base + KF * CH, TAIL)], rt_v.at[0], sem_i)
    idx_rows(0, 0, wait=True)
    pltpu.make_async_copy(s_hbm.at[pl.ds(base + KF * CH, TAIL)], st_v.at[0],
                          sem_i).wait()
    pltpu.make_async_copy(r_hbm.at[pl.ds(base + KF * CH, TAIL)], rt_v.at[0],
                          sem_i).wait()
    zero_cp.wait()
    plsc.subcore_barrier()
    pltpu.async_copy(xt_hbm.at[s_v.at[0, 0]], buf.at[0], sem)

    def emit_group(gb, g, is_last):
        if not is_last:
            idx_rows(g + 1, 1 - gb, wait=False)
        for u in range(G):
            cb = u % 2
            pltpu.make_async_copy(xt_hbm.at[s_v.at[gb, u]], buf.at[cb],
                                  sem).wait()

            def _wait_prev():
                pltpu.make_async_copy(buf.at[1 - cb], acc.at[r_v.at[gb, u]],
                                      sem_s).wait()
            if u == 0 and gb == 0 and not is_last:
                @pl.when(g > 0)
                def _():
                    _wait_prev()
            else:
                _wait_prev()

            if u < G - 1:
                pltpu.async_copy(xt_hbm.at[s_v.at[gb, u + 1]],
                                 buf.at[1 - cb], sem)
            elif not is_last:
                idx_rows(g + 1, 1 - gb, wait=True)
                pltpu.async_copy(xt_hbm.at[s_v.at[1 - gb, 0]],
                                 buf.at[1 - cb], sem)

            pltpu.async_copy(buf.at[cb], acc.at[r_v.at[gb, u]], sem_s,
                             add=True)

    def body(g2, carry):
        emit_group(0, g2 * 2, False)
        emit_group(1, g2 * 2 + 1, False)
        return carry

    lax.fori_loop(0, NG // 2, body, jnp.int32(0))
    emit_group(0, NG - 1, True)

    pltpu.async_copy(xt_hbm.at[st_v.at[0]], buf.at[0, pl.ds(0, TAIL)],
                     sem).wait()
    pltpu.make_async_copy(buf.at[1], acc.at[r_v.at[0, G - 1]], sem_s).wait()
    pltpu.sync_copy(buf.at[0, pl.ds(0, TAIL)], acc.at[rt_v.at[0]], add=True)
    plsc.subcore_barrier()

    rbase = sid * ZPT
    pltpu.sync_copy(acc.at[pl.ds(rbase, ZPT)],
                    out_hbm.at[cid, pl.ds(rbase, ZPT)])


def kernel(x, adj):
    s = adj[0].astype(jnp.int32)
    r = adj[1].astype(jnp.int32)
    zeros = jnp.zeros((ACC_ROWS, D), jnp.float32)
    xt = _logmap(x)
    parts = _agg(xt, s, r, zeros)
    return _expproj(parts)
